# baseline (speedup 1.0000x reference)
import functools

import jax
import jax.numpy as jnp
from jax import lax
from jax.experimental import pallas as pl
from jax.experimental.pallas import tpu as pltpu
from jax.experimental.pallas import tpu_sc as plsc

D = 1024
SCALE = 32.0

NC, NS, L = 2, 16, 16
NW = NC * NS

B = 16384
B_PER_W = B // NW
CB = 32
NCHUNK = B_PER_W // CB
SLICES_PER_ROW = D // L


def _scale_rows(rows_v, lo, hi):
    def row_body(r, c0):
        for c in range(SLICES_PER_ROW):
            sl = pl.ds(c * L, L)
            rows_v[r, sl] = rows_v[r, sl] * SCALE
        return c0

    lax.fori_loop(lo, hi, row_body, 0)


def _sc_embed(idx_hbm, table_hbm, out_hbm, idx_v, rows0, rows1, sg0, sg1,
              ss0, ss1):
    wid = lax.axis_index("s") * NC + lax.axis_index("c")
    base = wid * B_PER_W
    pltpu.sync_copy(idx_hbm.at[pl.ds(base, CB)], idx_v.at[pl.ds(0, CB)])

    bufs = (rows0, rows1)
    gsems = (sg0, sg1)
    ssems = (ss0, ss1)

    def gather_desc(g, b):
        return pltpu.make_async_copy(
            table_hbm.at[idx_v.at[pl.ds(g * CB, CB)]], bufs[b], gsems[b]
        )

    NH = 8
    H = CB // NH

    def scatter_desc(g, b, h):
        return pltpu.make_async_copy(
            bufs[b].at[pl.ds(h * H, H)],
            out_hbm.at[pl.ds(base + g * CB + h * H, H)],
            ssems[b],
        )

    gather_desc(0, 0).start()
    pltpu.sync_copy(
        idx_hbm.at[pl.ds(base + CB, B_PER_W - CB)],
        idx_v.at[pl.ds(CB, B_PER_W - CB)],
    )

    def chunk_pair(g0, carry):
        for bsel in range(2):
            g = g0 * 2 + bsel
            gather_desc(g, bsel).wait()
            @pl.when(g >= 1)
            def _():
                def wait_body(h, c0):
                    scatter_desc(g - 1, 1 - bsel, h).wait()
                    return c0

                lax.fori_loop(0, NH, wait_body, 0)

            @pl.when(g + 1 < NCHUNK)
            def _():
                gather_desc(g + 1, 1 - bsel).start()

            def quarter_body(h, c0):
                _scale_rows(bufs[bsel], h * H, (h + 1) * H)
                scatter_desc(g, bsel, h).start()
                return c0

            lax.fori_loop(0, NH, quarter_body, 0)
        return carry

    lax.fori_loop(0, NCHUNK // 2, chunk_pair, 0)

    def tail_wait(h, c0):
        scatter_desc(NCHUNK - 1, 1, h).wait()
        return c0

    lax.fori_loop(0, NH, tail_wait, 0)


@functools.partial(
    pl.kernel,
    mesh=plsc.VectorSubcoreMesh(core_axis_name="c", subcore_axis_name="s"),
    out_type=jax.ShapeDtypeStruct((B, D), jnp.float32),
    scratch_types=[
        pltpu.VMEM((B_PER_W,), jnp.int32),
        pltpu.VMEM((CB, D), jnp.float32),
        pltpu.VMEM((CB, D), jnp.float32),
        pltpu.SemaphoreType.DMA,
        pltpu.SemaphoreType.DMA,
        pltpu.SemaphoreType.DMA,
        pltpu.SemaphoreType.DMA,
    ],
)
def _embed_call(idx_hbm, table_hbm, out_hbm, idx_v, rows0, rows1, sg0, sg1,
                ss0, ss1):
    _sc_embed(idx_hbm, table_hbm, out_hbm, idx_v, rows0, rows1, sg0, sg1,
              ss0, ss1)


def kernel(input_ids, weight):
    idx = input_ids.reshape(-1).astype(jnp.int32)
    out = _embed_call(idx, weight)
    return out.reshape(input_ids.shape + (D,))


# speedup vs baseline: 1.0077x; 1.0077x over previous
import functools

import jax
import jax.numpy as jnp
from jax import lax
from jax.experimental import pallas as pl
from jax.experimental.pallas import tpu as pltpu
from jax.experimental.pallas import tpu_sc as plsc

D = 1024
SCALE = 32.0

NC, NS, L = 2, 16, 16
NW = NC * NS

B = 16384
B_PER_W = B // NW
CB = 32
NCHUNK = B_PER_W // CB
SLICES_PER_ROW = D // L
NBUF = 3
NH = 4
H = CB // NH


def _scale_rows(rows_v, lo, hi):
    def row_body(r, c0):
        for c in range(SLICES_PER_ROW):
            sl = pl.ds(c * L, L)
            rows_v[r, sl] = rows_v[r, sl] * SCALE
        return c0

    lax.fori_loop(lo, hi, row_body, 0)


def _sc_embed(idx_hbm, table_hbm, out_hbm, idx_v, rows0, rows1, rows2,
              sg0, sg1, sg2, ss0, ss1, ss2):
    wid = lax.axis_index("s") * NC + lax.axis_index("c")
    base = wid * B_PER_W
    pltpu.sync_copy(idx_hbm.at[pl.ds(base, CB)], idx_v.at[pl.ds(0, CB)])

    bufs = (rows0, rows1, rows2)
    gsems = (sg0, sg1, sg2)
    ssems = (ss0, ss1, ss2)

    def gather_desc(g, b):
        return pltpu.make_async_copy(
            table_hbm.at[idx_v.at[pl.ds(g * CB, CB)]], bufs[b], gsems[b]
        )

    def scatter_desc(g, b, h):
        return pltpu.make_async_copy(
            bufs[b].at[pl.ds(h * H, H)],
            out_hbm.at[pl.ds(base + g * CB + h * H, H)],
            ssems[b],
        )

    def scatter_waits(g, b):
        def wait_body(h, c0):
            scatter_desc(g, b, h).wait()
            return c0

        lax.fori_loop(0, NH, wait_body, 0)

    def scale_and_scatter(g, b):
        def quarter_body(h, c0):
            _scale_rows(bufs[b], h * H, (h + 1) * H)
            scatter_desc(g, b, h).start()
            return c0

        lax.fori_loop(0, NH, quarter_body, 0)

    gather_desc(0, 0).start()
    pltpu.sync_copy(
        idx_hbm.at[pl.ds(base + CB, B_PER_W - CB)],
        idx_v.at[pl.ds(CB, B_PER_W - CB)],
    )

    def chunk_tri(g0, carry):
        for bsel in range(NBUF):
            g = g0 * NBUF + bsel
            nxt = (bsel + 1) % NBUF
            @pl.when(g >= 2)
            def _():
                scatter_waits(g - 2, nxt)

            @pl.when(g + 1 < NCHUNK)
            def _():
                gather_desc(g + 1, nxt).start()

            gather_desc(g, bsel).wait()
            scale_and_scatter(g, bsel)
        return carry

    lax.fori_loop(0, (NCHUNK - 1) // NBUF, chunk_tri, 0)

    g_last = NCHUNK - 1
    b_last = g_last % NBUF
    gather_desc(g_last, b_last).wait()
    scale_and_scatter(g_last, b_last)

    for g in range(NCHUNK - NBUF, NCHUNK):
        scatter_waits(g, g % NBUF)


@functools.partial(
    pl.kernel,
    mesh=plsc.VectorSubcoreMesh(core_axis_name="c", subcore_axis_name="s"),
    out_type=jax.ShapeDtypeStruct((B, D), jnp.float32),
    scratch_types=[
        pltpu.VMEM((B_PER_W,), jnp.int32),
        pltpu.VMEM((CB, D), jnp.float32),
        pltpu.VMEM((CB, D), jnp.float32),
        pltpu.VMEM((CB, D), jnp.float32),
        pltpu.SemaphoreType.DMA,
        pltpu.SemaphoreType.DMA,
        pltpu.SemaphoreType.DMA,
        pltpu.SemaphoreType.DMA,
        pltpu.SemaphoreType.DMA,
        pltpu.SemaphoreType.DMA,
    ],
)
def _embed_call(idx_hbm, table_hbm, out_hbm, idx_v, rows0, rows1, rows2,
                sg0, sg1, sg2, ss0, ss1, ss2):
    _sc_embed(idx_hbm, table_hbm, out_hbm, idx_v, rows0, rows1, rows2,
              sg0, sg1, sg2, ss0, ss1, ss2)


def kernel(input_ids, weight):
    idx = input_ids.reshape(-1).astype(jnp.int32)
    out = _embed_call(idx, weight)
    return out.reshape(input_ids.shape + (D,))
